# baseline (speedup 1.0000x reference)
import jax
import jax.numpy as jnp
from jax.experimental import pallas as pl

_BB = 128


def _body(tok_ref, en_ref, w_ref, eb_ref, pe_ref, out_ref):
    e = jnp.dot(en_ref[:], w_ref[:], preferred_element_type=jnp.float32)
    out_ref[:, 0, :] = e + eb_ref[:]
    out_ref[:, 1:, :] = tok_ref[:] + pe_ref[:]


def kernel(tokens, energies, W, b, emb):
    batch, n_in, tsz = tokens.shape
    n_tok = emb.shape[0]
    pe = emb[1:].reshape(1, n_in, tsz)
    e_bias = (b + emb[0]).reshape(1, tsz)

    grid = (batch // _BB,)
    return pl.pallas_call(
        _body,
        grid=grid,
        in_specs=[
            pl.BlockSpec((_BB, n_in, tsz), lambda i: (i, 0, 0)),
            pl.BlockSpec((_BB, tsz), lambda i: (i, 0)),
            pl.BlockSpec((tsz, tsz), lambda i: (0, 0)),
            pl.BlockSpec((1, tsz), lambda i: (0, 0)),
            pl.BlockSpec((1, n_in, tsz), lambda i: (0, 0, 0)),
        ],
        out_specs=pl.BlockSpec((_BB, n_tok, tsz), lambda i: (i, 0, 0)),
        out_shape=jax.ShapeDtypeStruct((batch, n_tok, tsz), jnp.float32),
    )(tokens, energies, W, e_bias, pe)


# speedup vs baseline: 5.7632x; 5.7632x over previous
import jax
import jax.numpy as jnp
from jax.experimental import pallas as pl

_BL = 128


def _body(tok_ref, en_ref, wt_ref, eb_ref, pe_ref, out_ref):
    e = jnp.dot(wt_ref[:], en_ref[:], preferred_element_type=jnp.float32)
    out_ref[0, :, :] = e + eb_ref[:]
    out_ref[1:, :, :] = tok_ref[:] + pe_ref[:]


def kernel(tokens, energies, W, b, emb):
    batch, n_in, tsz = tokens.shape
    n_tok = emb.shape[0]
    tokens_t = tokens.transpose(1, 2, 0)
    energies_t = energies.T
    w_t = W.T
    pe = emb[1:].reshape(n_in, tsz, 1)
    eb = (b + emb[0]).reshape(tsz, 1)

    grid = (batch // _BL,)
    out_t = pl.pallas_call(
        _body,
        grid=grid,
        in_specs=[
            pl.BlockSpec((n_in, tsz, _BL), lambda j: (0, 0, j)),
            pl.BlockSpec((tsz, _BL), lambda j: (0, j)),
            pl.BlockSpec((tsz, tsz), lambda j: (0, 0)),
            pl.BlockSpec((tsz, 1), lambda j: (0, 0)),
            pl.BlockSpec((n_in, tsz, 1), lambda j: (0, 0, 0)),
        ],
        out_specs=pl.BlockSpec((n_tok, tsz, _BL), lambda j: (0, 0, j)),
        out_shape=jax.ShapeDtypeStruct((n_tok, tsz, batch), jnp.float32),
    )(tokens_t, energies_t, w_t, eb, pe)
    return out_t.transpose(2, 0, 1)


# speedup vs baseline: 6.0123x; 1.0432x over previous
import jax
import jax.numpy as jnp
from jax.experimental import pallas as pl

_BL = 256


def _body(tok_ref, en_ref, w_ref, eb_ref, pe_ref, out_ref):
    e = jax.lax.dot_general(
        w_ref[:], en_ref[:], (((0,), (0,)), ((), ())),
        preferred_element_type=jnp.float32)
    out_ref[0, :, :] = e + eb_ref[:]
    out_ref[1:, :, :] = tok_ref[:] + pe_ref[:]


def kernel(tokens, energies, W, b, emb):
    batch, n_in, tsz = tokens.shape
    n_tok = emb.shape[0]
    tokens_t = tokens.transpose(1, 2, 0)
    energies_t = energies.T
    pe = emb[1:].reshape(n_in, tsz, 1)
    eb = (b + emb[0]).reshape(tsz, 1)

    grid = (batch // _BL,)
    out_t = pl.pallas_call(
        _body,
        grid=grid,
        in_specs=[
            pl.BlockSpec((n_in, tsz, _BL), lambda j: (0, 0, j)),
            pl.BlockSpec((tsz, _BL), lambda j: (0, j)),
            pl.BlockSpec((tsz, tsz), lambda j: (0, 0)),
            pl.BlockSpec((tsz, 1), lambda j: (0, 0)),
            pl.BlockSpec((n_in, tsz, 1), lambda j: (0, 0, 0)),
        ],
        out_specs=pl.BlockSpec((n_tok, tsz, _BL), lambda j: (0, 0, j)),
        out_shape=jax.ShapeDtypeStruct((n_tok, tsz, batch), jnp.float32),
    )(tokens_t, energies_t, W, eb, pe)
    return out_t.transpose(2, 0, 1)
